# baseline (speedup 1.0000x reference)
import functools

import jax
import jax.numpy as jnp
from jax import lax
from jax.experimental import pallas as pl
from jax.experimental.pallas import tpu as pltpu
from jax.experimental.pallas import tpu_sc as plsc

EMBED = 2048
LANES = 16
W = 8
UNROLL = 8


@functools.cache
def _build(B, S):
    TOT = B * S
    NSTEP = TOT // W

    mesh = plsc.VectorSubcoreMesh(core_axis_name="c", subcore_axis_name="s")

    @functools.partial(
        pl.kernel,
        mesh=mesh,
        out_type=jax.ShapeDtypeStruct((TOT, EMBED), jnp.float32),
        scratch_types=[pltpu.VMEM((TOT,), jnp.int32)],
    )
    def emb_kernel(ids_hbm, table_hbm, pos_hbm, out_hbm, idx_all):
        pltpu.sync_copy(ids_hbm, idx_all)

        def body(idxs, pos_ref, out_ref):
            (i,) = idxs
            pltpu.sync_copy(
                table_hbm.at[idx_all.at[pl.ds(i * W, W)]], out_ref)

            @pl.loop(0, W)
            def _row(r):
                @pl.loop(0, EMBED, step=UNROLL * LANES)
                def _col(j):
                    slcs = [(pl.ds(r, 1), pl.ds(j + u * LANES, LANES))
                            for u in range(UNROLL)]
                    pvals = [pos_ref.at[slc][...] for slc in slcs]
                    for slc, p in zip(slcs, pvals):
                        plsc.addupdate(out_ref.at[slc], p)

        pltpu.emit_pipeline(
            body,
            grid=(NSTEP,),
            in_specs=[
                pl.BlockSpec((W, EMBED),
                             index_map=lambda i: (lax.rem(i, S // W), 0)),
            ],
            out_specs=[
                pl.BlockSpec((W, EMBED), index_map=lambda i: (i, 0)),
            ],
            core_axis_name=("c", "s"),
            dimension_semantics=(pltpu.PARALLEL,),
            _explicit_indices=True,
        )(pos_hbm, out_hbm)

    return emb_kernel


@jax.jit
def kernel(token_ids, token_table, pos_table):
    B, S = token_ids.shape
    ids_flat = token_ids.reshape(B * S).astype(jnp.int32)
    out = _build(B, S)(ids_flat, token_table, pos_table[:S])
    return out.reshape(B, S, EMBED)


# speedup vs baseline: 1.4875x; 1.4875x over previous
import functools

import jax
import jax.numpy as jnp
from jax import lax
from jax.experimental import pallas as pl
from jax.experimental.pallas import tpu as pltpu
from jax.experimental.pallas import tpu_sc as plsc

EMBED = 2048
LANES = 16
NC, NS = 2, 16
NW = NC * NS
SPC = 4
UNROLL = 8
NBUF = 3


@functools.cache
def _build(B, S):
    TOT = B * S
    CHUNK = SPC * B
    S_W = S // NW
    NCHUNK = S_W // SPC
    assert S % (NW * SPC) == 0

    mesh = plsc.VectorSubcoreMesh(core_axis_name="c", subcore_axis_name="s")

    scratch = [pltpu.VMEM((NCHUNK, CHUNK), jnp.int32)]
    for _ in range(NBUF):
        scratch.append(pltpu.VMEM((CHUNK, EMBED), jnp.float32))
        scratch.append(pltpu.VMEM((SPC, EMBED), jnp.float32))
        scratch.append(pltpu.SemaphoreType.DMA)
        scratch.append(pltpu.SemaphoreType.DMA)

    @functools.partial(
        pl.kernel,
        mesh=mesh,
        out_type=jax.ShapeDtypeStruct((TOT, EMBED), jnp.float32),
        scratch_types=scratch,
    )
    def emb_kernel(ids_hbm, table_hbm, pos_hbm, out_hbm, idx_v, *bufflat):
        wid = lax.axis_index("s") * NC + lax.axis_index("c")
        s_w0 = wid * S_W
        pltpu.sync_copy(ids_hbm.at[wid], idx_v)

        bufs = [tuple(bufflat[4 * k: 4 * k + 4]) for k in range(NBUF)]

        def issue_in(i, rows_v, pos_v, semi, semo):
            g = pltpu.async_copy(table_hbm.at[idx_v.at[i]], rows_v, semi)
            p = pltpu.async_copy(
                pos_hbm.at[pl.ds(s_w0 + i * SPC, SPC)], pos_v, semi)
            return g, p

        inflight = [None] * NBUF
        stores = [None] * NBUF
        for k in range(min(NBUF - 1, NCHUNK)):
            inflight[k] = issue_in(k, *bufs[k])
        for i in range(NCHUNK):
            b = i % NBUF
            if i + NBUF - 1 < NCHUNK:
                nb = (i + NBUF - 1) % NBUF
                if stores[nb] is not None:
                    for st in stores[nb]:
                        st.wait()
                    stores[nb] = None
                inflight[nb] = issue_in(i + NBUF - 1, *bufs[nb])
            g, p = inflight[b]
            g.wait()
            p.wait()
            rows_v, pos_v, _, semo = bufs[b]

            @pl.loop(0, CHUNK)
            def _row(r):
                so = lax.rem(r, SPC)

                @pl.loop(0, EMBED, step=UNROLL * LANES)
                def _col(j):
                    cols = [pl.ds(j + u * LANES, LANES) for u in range(UNROLL)]
                    pvals = [pos_v.at[(pl.ds(so, 1), c)][...] for c in cols]
                    for c, pv in zip(cols, pvals):
                        plsc.addupdate(rows_v.at[(pl.ds(r, 1), c)], pv)

            if stores[b] is not None:
                for st in stores[b]:
                    st.wait()
            stores[b] = [
                pltpu.async_copy(
                    rows_v.at[pl.ds(bb * SPC, SPC)],
                    out_hbm.at[pl.ds(bb * S + s_w0 + i * SPC, SPC)],
                    semo)
                for bb in range(B)
            ]
        for sts in stores:
            if sts is not None:
                for st in sts:
                    st.wait()

    return emb_kernel


@jax.jit
def kernel(token_ids, token_table, pos_table):
    B, S = token_ids.shape
    S_W = S // NW
    ids4 = token_ids.reshape(B, NW, S_W // SPC, SPC).astype(jnp.int32)
    ids_arr = ids4.transpose(1, 2, 0, 3).reshape(NW, S_W // SPC, SPC * B)
    out = _build(B, S)(ids_arr, token_table, pos_table[:S])
    return out.reshape(B, S, EMBED)


# speedup vs baseline: 1.6447x; 1.1057x over previous
import functools

import jax
import jax.numpy as jnp
from jax import lax
from jax.experimental import pallas as pl
from jax.experimental.pallas import tpu as pltpu
from jax.experimental.pallas import tpu_sc as plsc

EMBED = 2048
LANES = 16
NC, NS = 2, 16
NW = NC * NS
SPC = 4
UNROLL = 4
NBUF = 3


@functools.cache
def _build(B, S):
    TOT = B * S
    CHUNK = SPC * B
    S_W = S // NW
    NCHUNK = S_W // SPC
    assert S % (NW * SPC) == 0

    mesh = plsc.VectorSubcoreMesh(core_axis_name="c", subcore_axis_name="s")

    scratch = [pltpu.VMEM((NCHUNK, CHUNK), jnp.int32)]
    for _ in range(NBUF):
        scratch.append(pltpu.VMEM((CHUNK, EMBED), jnp.float32))
        scratch.append(pltpu.VMEM((SPC, EMBED), jnp.float32))
        scratch.append(pltpu.SemaphoreType.DMA)
        scratch.append(pltpu.SemaphoreType.DMA)

    @functools.partial(
        pl.kernel,
        mesh=mesh,
        out_type=jax.ShapeDtypeStruct((TOT, EMBED), jnp.float32),
        scratch_types=scratch,
    )
    def emb_kernel(ids_hbm, table_hbm, pos_hbm, out_hbm, idx_v, *bufflat):
        wid = lax.axis_index("s") * NC + lax.axis_index("c")
        s_w0 = wid * S_W
        pltpu.sync_copy(ids_hbm.at[wid], idx_v)

        bufs = [tuple(bufflat[4 * k: 4 * k + 4]) for k in range(NBUF)]

        def issue_in(i, rows_v, pos_v, semi, semo):
            g = pltpu.async_copy(table_hbm.at[idx_v.at[i]], rows_v, semi)
            p = pltpu.async_copy(
                pos_hbm.at[pl.ds(s_w0 + i * SPC, SPC)], pos_v, semi)
            return g, p

        inflight = [None] * NBUF
        stores = [None] * NBUF
        for k in range(min(NBUF - 1, NCHUNK)):
            inflight[k] = issue_in(k, *bufs[k])
        for i in range(NCHUNK):
            b = i % NBUF
            if i + NBUF - 1 < NCHUNK:
                nb = (i + NBUF - 1) % NBUF
                if stores[nb] is not None:
                    for st in stores[nb]:
                        st.wait()
                    stores[nb] = None
                inflight[nb] = issue_in(i + NBUF - 1, *bufs[nb])
            g, p = inflight[b]
            g.wait()
            p.wait()
            rows_v, pos_v, _, semo = bufs[b]

            @pl.loop(0, SPC)
            def _so(so):
                @pl.loop(0, EMBED, step=UNROLL * LANES)
                def _col(j):
                    cols = [pl.ds(j + u * LANES, LANES) for u in range(UNROLL)]
                    pvals = [pos_v.at[(pl.ds(so, 1), c)][...] for c in cols]
                    for bb in range(B):
                        for c, pv in zip(cols, pvals):
                            plsc.addupdate(
                                rows_v.at[(pl.ds(so + bb * SPC, 1), c)], pv)

            if stores[b] is not None:
                for st in stores[b]:
                    st.wait()
            stores[b] = [
                pltpu.async_copy(
                    rows_v.at[pl.ds(bb * SPC, SPC)],
                    out_hbm.at[pl.ds(bb * S + s_w0 + i * SPC, SPC)],
                    semo)
                for bb in range(B)
            ]
        for sts in stores:
            if sts is not None:
                for st in sts:
                    st.wait()

    return emb_kernel


@jax.jit
def kernel(token_ids, token_table, pos_table):
    B, S = token_ids.shape
    S_W = S // NW
    ids4 = token_ids.reshape(B, NW, S_W // SPC, SPC).astype(jnp.int32)
    ids_arr = ids4.transpose(1, 2, 0, 3).reshape(NW, S_W // SPC, SPC * B)
    out = _build(B, S)(ids_arr, token_table, pos_table[:S])
    return out.reshape(B, S, EMBED)
